# baseline (speedup 1.0000x reference)
import functools

import jax
import jax.numpy as jnp
from jax import lax
from jax.experimental import pallas as pl
from jax.experimental.pallas import tpu as pltpu
from jax.experimental.pallas import tpu_sc as plsc

F32 = jnp.float32
LANES = 128
BATCH = 32
RING = 4
BN = 400


def _sc_counts(dst2, z128, ones128, n):
    nb = dst2.shape[0] // 16
    half = (nb + 1) // 2
    n_acc = z128.shape[0] * 16
    zr = n_acc // 16
    cr = (n // 16) // 8 * 8
    tail = n - cr * 16

    mesh = plsc.VectorSubcoreMesh(core_axis_name="c", subcore_axis_name="s")

    def body(dst_ref, z_ref, ones_ref, cnt_ref, dst_v, ones_v, cacc):
        cid = lax.axis_index("c")
        sid = lax.axis_index("s")
        pltpu.sync_copy(dst_ref.at[pl.ds(sid * nb, nb)], dst_v)
        pltpu.sync_copy(ones_ref, ones_v)
        pltpu.sync_copy(z_ref, cacc.at[pl.ds(sid * zr, zr)])
        plsc.subcore_barrier()

        def cbody(b, carry):
            pltpu.sync_copy(ones_v, cacc.at[dst_v.at[b]], add=True)
            return carry
        lax.fori_loop(cid * half, half + cid * (nb - half), cbody, 0)
        plsc.subcore_barrier()
        pltpu.sync_copy(cacc.at[pl.ds(sid * cr, cr)],
                        cnt_ref.at[cid].at[pl.ds(sid * cr, cr)])
        if tail:
            @pl.when(sid == 0)
            def _():
                pltpu.sync_copy(cacc.at[pl.ds(cr * 16, tail)],
                                cnt_ref.at[cid].at[pl.ds(cr * 16, tail)])

    fn = pl.kernel(
        body,
        out_type=jax.ShapeDtypeStruct((2, n, LANES), F32),
        mesh=mesh,
        scratch_types=[
            pltpu.VMEM((nb, BATCH), jnp.int32),
            pltpu.VMEM((BATCH, LANES), F32),
            pltpu.VMEM_SHARED((n_acc, LANES), F32),
        ])
    return fn(dst2, z128, ones128)


def _sc_segsum(u, src2, dst2, z128, nch):
    n = u.shape[1]
    nb = src2.shape[0] // 16
    n_acc = z128.shape[0] * 16
    zr = n_acc // 16
    cr = (n // 16) // 8 * 8
    tail = n - cr * 16

    mesh = plsc.VectorSubcoreMesh(core_axis_name="c", subcore_axis_name="s")

    nbh = nb // 8

    def body(u_ref, src_ref, dst_ref, z128_ref, agg_ref,
             src_v, dst_v, *rest):
        bufs = rest[:RING]
        acc = rest[RING]
        sems = rest[RING + 1:]
        cid = lax.axis_index("c")
        sid = lax.axis_index("s")

        for ch in range(nch):
            @pl.when(cid == (ch % 2))
            def _():
                pltpu.sync_copy(z128_ref, acc.at[pl.ds(sid * zr, zr)])
                plsc.subcore_barrier()
                for h in range(8):
                    base = sid * nb + h * nbh
                    pltpu.sync_copy(src_ref.at[pl.ds(base, nbh)], src_v)
                    pltpu.sync_copy(dst_ref.at[pl.ds(base, nbh)], dst_v)
                    for j in range(RING):
                        pltpu.make_async_copy(u_ref.at[ch].at[src_v.at[j]],
                                              bufs[j], sems[j]).start()

                    @pl.loop(0, nbh, step=RING)
                    def _(i):
                        for j in range(RING):
                            b = i + j
                            pltpu.make_async_copy(
                                u_ref.at[ch].at[src_v.at[b]],
                                bufs[j], sems[j]).wait()
                            pltpu.sync_copy(bufs[j], acc.at[dst_v.at[b]],
                                            add=True)

                            @pl.when(b + RING < nbh)
                            def _():
                                pltpu.make_async_copy(
                                    u_ref.at[ch].at[src_v.at[b + RING]],
                                    bufs[j], sems[j]).start()
                plsc.subcore_barrier()
                pltpu.sync_copy(acc.at[pl.ds(sid * cr, cr)],
                                agg_ref.at[ch].at[pl.ds(sid * cr, cr)])
                if tail:
                    @pl.when(sid == 0)
                    def _():
                        pltpu.sync_copy(acc.at[pl.ds(cr * 16, tail)],
                                        agg_ref.at[ch].at[pl.ds(cr * 16, tail)])
                plsc.subcore_barrier()

    fn = pl.kernel(
        body,
        out_type=jax.ShapeDtypeStruct((nch, n, LANES), F32),
        mesh=mesh,
        scratch_types=(
            [pltpu.VMEM((nb // 8, BATCH), jnp.int32),
             pltpu.VMEM((nb // 8, BATCH), jnp.int32)]
            + [pltpu.VMEM((BATCH, LANES), F32)] * RING
            + [pltpu.VMEM_SHARED((n_acc, LANES), F32)]
            + [pltpu.SemaphoreType.DMA] * RING))
    return fn(u, src2, dst2, z128)


def _dot_chunks(chunks, w_ref, m):
    acc = None
    for j, a in enumerate(chunks):
        p = jnp.dot(a.astype(jnp.bfloat16),
                    w_ref[pl.ds(j * LANES, LANES), :].astype(jnp.bfloat16),
                    preferred_element_type=F32)
        acc = p if acc is None else acc + p
    return acc


def _inv_deg(cnt_ref):
    cnt = cnt_ref[0, :, 0] + cnt_ref[1, :, 0]
    return (1.0 / jnp.maximum(cnt, 1.0))[:, None]


def _mm_mean(aggc, cntp, vc, w_t, b):
    kc, n, _ = aggc.shape
    m = w_t.shape[1]
    mc = m // LANES

    def body(agg_ref, cnt_ref, v_ref, w_ref, b_ref, o_ref):
        inv = _inv_deg(cnt_ref)
        chunks = [agg_ref[j] * inv for j in range(kc)]
        acc = _dot_chunks(chunks, w_ref, m)
        for jj in range(mc):
            o_ref[jj] = jnp.maximum(
                acc[:, jj * LANES:(jj + 1) * LANES] + v_ref[jj]
                + b_ref[0][None, jj * LANES:(jj + 1) * LANES], 0.0)

    return pl.pallas_call(
        body,
        grid=(n // BN,),
        in_specs=[
            pl.BlockSpec((kc, BN, LANES), lambda i: (0, i, 0)),
            pl.BlockSpec((2, BN, LANES), lambda i: (0, i, 0)),
            pl.BlockSpec((mc, BN, LANES), lambda i: (0, i, 0)),
            pl.BlockSpec(w_t.shape, lambda i: (0, 0)),
            pl.BlockSpec(b.shape, lambda i: (0, 0)),
        ],
        out_specs=pl.BlockSpec((mc, BN, LANES), lambda i: (0, i, 0)),
        out_shape=jax.ShapeDtypeStruct((mc, n, LANES), F32),
    )(aggc, cntp, vc, w_t, b)


def _mm_chunked(ac, w_t):
    kc, n, _ = ac.shape
    m = w_t.shape[1]
    mc = m // LANES

    def body(a_ref, w_ref, o_ref):
        acc = _dot_chunks([a_ref[j] for j in range(kc)], w_ref, m)
        for jj in range(mc):
            o_ref[jj] = acc[:, jj * LANES:(jj + 1) * LANES]

    return pl.pallas_call(
        body,
        grid=(n // BN,),
        in_specs=[
            pl.BlockSpec((kc, BN, LANES), lambda i: (0, i, 0)),
            pl.BlockSpec(w_t.shape, lambda i: (0, 0)),
        ],
        out_specs=pl.BlockSpec((mc, BN, LANES), lambda i: (0, i, 0)),
        out_shape=jax.ShapeDtypeStruct((mc, n, LANES), F32),
    )(ac, w_t)


def _combine(aggc, vc, cntp, b):
    nch, n, _ = aggc.shape

    def body(agg_ref, v_ref, cnt_ref, b_ref, o_ref):
        inv = _inv_deg(cnt_ref)
        for j in range(nch):
            o_ref[j] = jnp.maximum(
                agg_ref[j] * inv + v_ref[j] + b_ref[0][None, j * LANES:(j + 1) * LANES],
                0.0)

    return pl.pallas_call(
        body,
        grid=(n // BN,),
        in_specs=[
            pl.BlockSpec((nch, BN, LANES), lambda i: (0, i, 0)),
            pl.BlockSpec((nch, BN, LANES), lambda i: (0, i, 0)),
            pl.BlockSpec((2, BN, LANES), lambda i: (0, i, 0)),
            pl.BlockSpec(b.shape, lambda i: (0, 0)),
        ],
        out_specs=pl.BlockSpec((nch, BN, LANES), lambda i: (0, i, 0)),
        out_shape=jax.ShapeDtypeStruct((nch, n, LANES), F32),
    )(aggc, vc, cntp, b)


def _combine_att(aggc, vc, cntp, b, wa, ba):
    nch, n, _ = aggc.shape
    nblocks = n // BN

    def body(agg_ref, v_ref, cnt_ref, b_ref, wa_ref, ba_ref,
             h_ref, s_ref, mz_ref, m_sc, z_sc):
        i = pl.program_id(0)

        @pl.when(i == 0)
        def _():
            m_sc[0] = -jnp.inf
            z_sc[0] = 0.0

        inv = _inv_deg(cnt_ref)
        sdot = None
        for j in range(nch):
            h = jnp.maximum(
                agg_ref[j] * inv + v_ref[j] + b_ref[0][None, j * LANES:(j + 1) * LANES],
                0.0)
            h_ref[j] = h
            p = jnp.sum(h * wa_ref[0][None, j * LANES:(j + 1) * LANES], axis=1)
            sdot = p if sdot is None else sdot + p
        s = jnp.tanh(sdot + ba_ref[0, 0])
        s_ref[...] = s[:, None]

        m_old = m_sc[0]
        m_new = jnp.maximum(m_old, jnp.max(s))
        z_sc[0] = z_sc[0] * jnp.exp(m_old - m_new) + jnp.sum(jnp.exp(s - m_new))
        m_sc[0] = m_new

        @pl.when(i == nblocks - 1)
        def _():
            mz_ref[0, 0] = m_sc[0]
            mz_ref[1, 0] = z_sc[0]

    return pl.pallas_call(
        body,
        grid=(nblocks,),
        in_specs=[
            pl.BlockSpec((nch, BN, LANES), lambda i: (0, i, 0)),
            pl.BlockSpec((nch, BN, LANES), lambda i: (0, i, 0)),
            pl.BlockSpec((2, BN, LANES), lambda i: (0, i, 0)),
            pl.BlockSpec(b.shape, lambda i: (0, 0)),
            pl.BlockSpec(wa.shape, lambda i: (0, 0)),
            pl.BlockSpec((1, 1), lambda i: (0, 0)),
        ],
        out_specs=[
            pl.BlockSpec((nch, BN, LANES), lambda i: (0, i, 0)),
            pl.BlockSpec((BN, 1), lambda i: (i, 0)),
            pl.BlockSpec(memory_space=pltpu.SMEM),
        ],
        out_shape=[
            jax.ShapeDtypeStruct((nch, n, LANES), F32),
            jax.ShapeDtypeStruct((n, 1), F32),
            jax.ShapeDtypeStruct((2, 1), F32),
        ],
        scratch_shapes=[pltpu.SMEM((1,), F32), pltpu.SMEM((1,), F32)],
    )(aggc, vc, cntp, b, wa, ba)


def _final(h3c, s, mz, wf_t, bf):
    nch, n, _ = h3c.shape
    m = wf_t.shape[1]

    def body(h_ref, s_ref, mz_ref, w_ref, b_ref, o_ref):
        att = jnp.exp(s_ref[...] - mz_ref[0, 0]) / mz_ref[1, 0]
        chunks = [h_ref[j] * att for j in range(nch)]
        o_ref[...] = _dot_chunks(chunks, w_ref, m) + b_ref[0][None, :]

    return pl.pallas_call(
        body,
        grid=(n // BN,),
        in_specs=[
            pl.BlockSpec((nch, BN, LANES), lambda i: (0, i, 0)),
            pl.BlockSpec((BN, 1), lambda i: (i, 0)),
            pl.BlockSpec(memory_space=pltpu.SMEM),
            pl.BlockSpec(wf_t.shape, lambda i: (0, 0)),
            pl.BlockSpec(bf.shape, lambda i: (0, 0)),
        ],
        out_specs=pl.BlockSpec((BN, m), lambda i: (i, 0)),
        out_shape=jax.ShapeDtypeStruct((n, m), F32),
    )(h3c, s, mz, wf_t, bf)


def kernel(x, edge_index, Wl0, bl0, Wr0, Wl1, bl1, Wr1, Wl2, bl2, Wr2,
           Wa, ba, Wf, bf):
    n, d_in = x.shape
    e = edge_index.shape[1]

    ept = -(-e // (16 * 8 * LANES)) * 8 * LANES
    e_pad = 16 * ept
    pad = e_pad - e
    src = edge_index[0]
    dst = edge_index[1]
    if pad:
        src = jnp.concatenate([src, jnp.zeros((pad,), jnp.int32)])
        dst = jnp.concatenate(
            [dst, n + (jnp.arange(pad, dtype=jnp.int32) % 16)])
    src2 = src.reshape(-1, BATCH)
    dst2 = dst.reshape(-1, BATCH)

    n_acc = -(-(n + 16) // 128) * 128
    z128 = jnp.zeros((n_acc // 16, LANES), F32)
    ones128 = jnp.ones((BATCH, LANES), F32)

    bf16 = jnp.bfloat16
    wl0_t = Wl0.T.astype(bf16)
    wr0_t = Wr0.T.astype(bf16)
    wl1_t = Wl1.T.astype(bf16)
    wr1_t = Wr1.T.astype(bf16)
    wl2_t = Wl2.T.astype(bf16)
    wr2_t = Wr2.T.astype(bf16)
    wf_t = Wf.T.astype(bf16)
    b0 = bl0.reshape(1, -1)
    b1 = bl1.reshape(1, -1)
    b2 = bl2.reshape(1, -1)
    bfr = bf.reshape(1, -1)
    bar = ba.reshape(1, 1)

    xc = x.reshape(n, d_in // LANES, LANES).transpose(1, 0, 2)

    cntp = _sc_counts(dst2, z128, ones128, n)
    agg0 = _sc_segsum(xc, src2, dst2, z128, nch=d_in // LANES)
    v0 = _mm_chunked(xc, wr0_t)
    h1 = _mm_mean(agg0, cntp, v0, wl0_t, b0)

    u1 = _mm_chunked(h1, wl1_t)
    agg1 = _sc_segsum(u1, src2, dst2, z128, nch=Wl1.shape[0] // LANES)
    v1 = _mm_chunked(h1, wr1_t)
    h2 = _combine(agg1, v1, cntp, b1)

    u2 = _mm_chunked(h2, wl2_t)
    agg2 = _sc_segsum(u2, src2, dst2, z128, nch=Wl2.shape[0] // LANES)
    v2 = _mm_chunked(h2, wr2_t)
    h3, s, mz = _combine_att(agg2, v2, cntp, b2, Wa, bar)

    return _final(h3, s, mz, wf_t, bfr)


# speedup vs baseline: 1.0440x; 1.0440x over previous
import functools

import jax
import jax.numpy as jnp
from jax import lax
from jax.experimental import pallas as pl
from jax.experimental.pallas import tpu as pltpu
from jax.experimental.pallas import tpu_sc as plsc

F32 = jnp.float32
LANES = 128
BATCH = 64
RING = 2
BN = 400


def _sc_counts(dst2, z128, ones128, n):
    nb = dst2.shape[0] // 16
    half = (nb + 1) // 2
    n_acc = z128.shape[0] * 16
    zr = n_acc // 16
    cr = (n // 16) // 8 * 8
    tail = n - cr * 16

    mesh = plsc.VectorSubcoreMesh(core_axis_name="c", subcore_axis_name="s")

    def body(dst_ref, z_ref, ones_ref, cnt_ref, dst_v, ones_v, cacc):
        cid = lax.axis_index("c")
        sid = lax.axis_index("s")
        pltpu.sync_copy(dst_ref.at[pl.ds(sid * nb, nb)], dst_v)
        pltpu.sync_copy(ones_ref, ones_v)
        pltpu.sync_copy(z_ref, cacc.at[pl.ds(sid * zr, zr)])
        plsc.subcore_barrier()

        def cbody(b, carry):
            pltpu.sync_copy(ones_v, cacc.at[dst_v.at[b]], add=True)
            return carry
        lax.fori_loop(cid * half, half + cid * (nb - half), cbody, 0)
        plsc.subcore_barrier()
        pltpu.sync_copy(cacc.at[pl.ds(sid * cr, cr)],
                        cnt_ref.at[cid].at[pl.ds(sid * cr, cr)])
        if tail:
            @pl.when(sid == 0)
            def _():
                pltpu.sync_copy(cacc.at[pl.ds(cr * 16, tail)],
                                cnt_ref.at[cid].at[pl.ds(cr * 16, tail)])

    fn = pl.kernel(
        body,
        out_type=jax.ShapeDtypeStruct((2, n, LANES), F32),
        mesh=mesh,
        scratch_types=[
            pltpu.VMEM((nb, BATCH), jnp.int32),
            pltpu.VMEM((BATCH, LANES), F32),
            pltpu.VMEM_SHARED((n_acc, LANES), F32),
        ])
    return fn(dst2, z128, ones128)


def _sc_segsum(u, src2, dst2, z128, nch):
    n = u.shape[1]
    nb = src2.shape[0] // 16
    n_acc = z128.shape[0] * 16
    zr = n_acc // 16
    cr = (n // 16) // 8 * 8
    tail = n - cr * 16

    mesh = plsc.VectorSubcoreMesh(core_axis_name="c", subcore_axis_name="s")

    nbh = nb // 2

    def body(u_ref, src_ref, dst_ref, z128_ref, agg_ref,
             src_v, dst_v, *rest):
        bufs = rest[:RING]
        acc = rest[RING]
        sems = rest[RING + 1:]
        cid = lax.axis_index("c")
        sid = lax.axis_index("s")

        for ch in range(nch):
            @pl.when(cid == (ch % 2))
            def _():
                pltpu.sync_copy(z128_ref, acc.at[pl.ds(sid * zr, zr)])
                plsc.subcore_barrier()
                for h in range(2):
                    base = sid * nb + h * nbh
                    pltpu.sync_copy(src_ref.at[pl.ds(base, nbh)], src_v)
                    pltpu.sync_copy(dst_ref.at[pl.ds(base, nbh)], dst_v)
                    for j in range(RING):
                        pltpu.make_async_copy(u_ref.at[ch].at[src_v.at[j]],
                                              bufs[j], sems[j]).start()

                    @pl.loop(0, nbh, step=RING)
                    def _(i):
                        for j in range(RING):
                            b = i + j
                            pltpu.make_async_copy(
                                u_ref.at[ch].at[src_v.at[b]],
                                bufs[j], sems[j]).wait()
                            pltpu.sync_copy(bufs[j], acc.at[dst_v.at[b]],
                                            add=True)

                            @pl.when(b + RING < nbh)
                            def _():
                                pltpu.make_async_copy(
                                    u_ref.at[ch].at[src_v.at[b + RING]],
                                    bufs[j], sems[j]).start()
                plsc.subcore_barrier()
                pltpu.sync_copy(acc.at[pl.ds(sid * cr, cr)],
                                agg_ref.at[ch].at[pl.ds(sid * cr, cr)])
                if tail:
                    @pl.when(sid == 0)
                    def _():
                        pltpu.sync_copy(acc.at[pl.ds(cr * 16, tail)],
                                        agg_ref.at[ch].at[pl.ds(cr * 16, tail)])
                plsc.subcore_barrier()

    fn = pl.kernel(
        body,
        out_type=jax.ShapeDtypeStruct((nch, n, LANES), F32),
        mesh=mesh,
        scratch_types=(
            [pltpu.VMEM((nb // 2, BATCH), jnp.int32),
             pltpu.VMEM((nb // 2, BATCH), jnp.int32)]
            + [pltpu.VMEM((BATCH, LANES), F32)] * RING
            + [pltpu.VMEM_SHARED((n_acc, LANES), F32)]
            + [pltpu.SemaphoreType.DMA] * RING))
    return fn(u, src2, dst2, z128)


def _dot_chunks(chunks, w_ref, m):
    acc = None
    for j, a in enumerate(chunks):
        p = jnp.dot(a.astype(jnp.bfloat16),
                    w_ref[pl.ds(j * LANES, LANES), :].astype(jnp.bfloat16),
                    preferred_element_type=F32)
        acc = p if acc is None else acc + p
    return acc


def _inv_deg(cnt_ref):
    cnt = cnt_ref[0, :, 0] + cnt_ref[1, :, 0]
    return (1.0 / jnp.maximum(cnt, 1.0))[:, None]


def _mm_mean(aggc, cntp, vc, w_t, b):
    kc, n, _ = aggc.shape
    m = w_t.shape[1]
    mc = m // LANES

    def body(agg_ref, cnt_ref, v_ref, w_ref, b_ref, o_ref):
        inv = _inv_deg(cnt_ref)
        chunks = [agg_ref[j] * inv for j in range(kc)]
        acc = _dot_chunks(chunks, w_ref, m)
        for jj in range(mc):
            o_ref[jj] = jnp.maximum(
                acc[:, jj * LANES:(jj + 1) * LANES]
                + v_ref[jj].astype(F32)
                + b_ref[0][None, jj * LANES:(jj + 1) * LANES],
                0.0).astype(jnp.bfloat16)

    return pl.pallas_call(
        body,
        grid=(n // BN,),
        in_specs=[
            pl.BlockSpec((kc, BN, LANES), lambda i: (0, i, 0)),
            pl.BlockSpec((2, BN, LANES), lambda i: (0, i, 0)),
            pl.BlockSpec((mc, BN, LANES), lambda i: (0, i, 0)),
            pl.BlockSpec(w_t.shape, lambda i: (0, 0)),
            pl.BlockSpec(b.shape, lambda i: (0, 0)),
        ],
        out_specs=pl.BlockSpec((mc, BN, LANES), lambda i: (0, i, 0)),
        out_shape=jax.ShapeDtypeStruct((mc, n, LANES), jnp.bfloat16),
    )(aggc, cntp, vc, w_t, b)


def _mm_chunked(ac, w_t, out_dtype=F32):
    kc, n, _ = ac.shape
    m = w_t.shape[1]
    mc = m // LANES

    def body(a_ref, w_ref, o_ref):
        acc = _dot_chunks([a_ref[j] for j in range(kc)], w_ref, m)
        acc = acc.astype(out_dtype)
        for jj in range(mc):
            o_ref[jj] = acc[:, jj * LANES:(jj + 1) * LANES]

    return pl.pallas_call(
        body,
        grid=(n // BN,),
        in_specs=[
            pl.BlockSpec((kc, BN, LANES), lambda i: (0, i, 0)),
            pl.BlockSpec(w_t.shape, lambda i: (0, 0)),
        ],
        out_specs=pl.BlockSpec((mc, BN, LANES), lambda i: (0, i, 0)),
        out_shape=jax.ShapeDtypeStruct((mc, n, LANES), out_dtype),
    )(ac, w_t)


def _combine(aggc, vc, cntp, b):
    nch, n, _ = aggc.shape

    def body(agg_ref, v_ref, cnt_ref, b_ref, o_ref):
        inv = _inv_deg(cnt_ref)
        for j in range(nch):
            o_ref[j] = jnp.maximum(
                agg_ref[j] * inv + v_ref[j].astype(F32)
                + b_ref[0][None, j * LANES:(j + 1) * LANES],
                0.0).astype(jnp.bfloat16)

    return pl.pallas_call(
        body,
        grid=(n // BN,),
        in_specs=[
            pl.BlockSpec((nch, BN, LANES), lambda i: (0, i, 0)),
            pl.BlockSpec((nch, BN, LANES), lambda i: (0, i, 0)),
            pl.BlockSpec((2, BN, LANES), lambda i: (0, i, 0)),
            pl.BlockSpec(b.shape, lambda i: (0, 0)),
        ],
        out_specs=pl.BlockSpec((nch, BN, LANES), lambda i: (0, i, 0)),
        out_shape=jax.ShapeDtypeStruct((nch, n, LANES), jnp.bfloat16),
    )(aggc, vc, cntp, b)


def _combine_att(aggc, vc, cntp, b, wa, ba):
    nch, n, _ = aggc.shape
    nblocks = n // BN

    def body(agg_ref, v_ref, cnt_ref, b_ref, wa_ref, ba_ref,
             h_ref, s_ref, mz_ref, m_sc, z_sc):
        i = pl.program_id(0)

        @pl.when(i == 0)
        def _():
            m_sc[0] = -jnp.inf
            z_sc[0] = 0.0

        inv = _inv_deg(cnt_ref)
        sdot = None
        for j in range(nch):
            h = jnp.maximum(
                agg_ref[j] * inv + v_ref[j].astype(F32)
                + b_ref[0][None, j * LANES:(j + 1) * LANES],
                0.0)
            h_ref[j] = h
            p = jnp.sum(h * wa_ref[0][None, j * LANES:(j + 1) * LANES], axis=1)
            sdot = p if sdot is None else sdot + p
        s = jnp.tanh(sdot + ba_ref[0, 0])
        s_ref[...] = s[:, None]

        m_old = m_sc[0]
        m_new = jnp.maximum(m_old, jnp.max(s))
        z_sc[0] = z_sc[0] * jnp.exp(m_old - m_new) + jnp.sum(jnp.exp(s - m_new))
        m_sc[0] = m_new

        @pl.when(i == nblocks - 1)
        def _():
            mz_ref[0, 0] = m_sc[0]
            mz_ref[1, 0] = z_sc[0]

    return pl.pallas_call(
        body,
        grid=(nblocks,),
        in_specs=[
            pl.BlockSpec((nch, BN, LANES), lambda i: (0, i, 0)),
            pl.BlockSpec((nch, BN, LANES), lambda i: (0, i, 0)),
            pl.BlockSpec((2, BN, LANES), lambda i: (0, i, 0)),
            pl.BlockSpec(b.shape, lambda i: (0, 0)),
            pl.BlockSpec(wa.shape, lambda i: (0, 0)),
            pl.BlockSpec((1, 1), lambda i: (0, 0)),
        ],
        out_specs=[
            pl.BlockSpec((nch, BN, LANES), lambda i: (0, i, 0)),
            pl.BlockSpec((BN, 1), lambda i: (i, 0)),
            pl.BlockSpec(memory_space=pltpu.SMEM),
        ],
        out_shape=[
            jax.ShapeDtypeStruct((nch, n, LANES), F32),
            jax.ShapeDtypeStruct((n, 1), F32),
            jax.ShapeDtypeStruct((2, 1), F32),
        ],
        scratch_shapes=[pltpu.SMEM((1,), F32), pltpu.SMEM((1,), F32)],
    )(aggc, vc, cntp, b, wa, ba)


def _final(h3c, s, mz, wf_t, bf):
    nch, n, _ = h3c.shape
    m = wf_t.shape[1]

    def body(h_ref, s_ref, mz_ref, w_ref, b_ref, o_ref):
        att = jnp.exp(s_ref[...] - mz_ref[0, 0]) / mz_ref[1, 0]
        chunks = [h_ref[j] * att for j in range(nch)]
        o_ref[...] = _dot_chunks(chunks, w_ref, m) + b_ref[0][None, :]

    return pl.pallas_call(
        body,
        grid=(n // BN,),
        in_specs=[
            pl.BlockSpec((nch, BN, LANES), lambda i: (0, i, 0)),
            pl.BlockSpec((BN, 1), lambda i: (i, 0)),
            pl.BlockSpec(memory_space=pltpu.SMEM),
            pl.BlockSpec(wf_t.shape, lambda i: (0, 0)),
            pl.BlockSpec(bf.shape, lambda i: (0, 0)),
        ],
        out_specs=pl.BlockSpec((BN, m), lambda i: (i, 0)),
        out_shape=jax.ShapeDtypeStruct((n, m), F32),
    )(h3c, s, mz, wf_t, bf)


def kernel(x, edge_index, Wl0, bl0, Wr0, Wl1, bl1, Wr1, Wl2, bl2, Wr2,
           Wa, ba, Wf, bf):
    n, d_in = x.shape
    e = edge_index.shape[1]

    ept = -(-e // (16 * 8 * LANES)) * 8 * LANES
    e_pad = 16 * ept
    pad = e_pad - e
    src = edge_index[0]
    dst = edge_index[1]
    if pad:
        src = jnp.concatenate([src, jnp.zeros((pad,), jnp.int32)])
        dst = jnp.concatenate(
            [dst, n + (jnp.arange(pad, dtype=jnp.int32) % 16)])
    src2 = src.reshape(-1, BATCH)
    dst2 = dst.reshape(-1, BATCH)

    n_acc = -(-(n + 16) // 128) * 128
    z128 = jnp.zeros((n_acc // 16, LANES), F32)
    ones128 = jnp.ones((BATCH, LANES), F32)

    bf16 = jnp.bfloat16
    wl0_t = Wl0.T.astype(bf16)
    wr0_t = Wr0.T.astype(bf16)
    wl1_t = Wl1.T.astype(bf16)
    wr1_t = Wr1.T.astype(bf16)
    wl2_t = Wl2.T.astype(bf16)
    wr2_t = Wr2.T.astype(bf16)
    wf_t = Wf.T.astype(bf16)
    b0 = bl0.reshape(1, -1)
    b1 = bl1.reshape(1, -1)
    b2 = bl2.reshape(1, -1)
    bfr = bf.reshape(1, -1)
    bar = ba.reshape(1, 1)

    xc = x.reshape(n, d_in // LANES, LANES).transpose(1, 0, 2)

    cntp = _sc_counts(dst2, z128, ones128, n)
    agg0 = _sc_segsum(xc, src2, dst2, z128, nch=d_in // LANES)
    v0 = _mm_chunked(xc, wr0_t, jnp.bfloat16)
    h1 = _mm_mean(agg0, cntp, v0, wl0_t, b0)

    u1 = _mm_chunked(h1, wl1_t)
    agg1 = _sc_segsum(u1, src2, dst2, z128, nch=Wl1.shape[0] // LANES)
    v1 = _mm_chunked(h1, wr1_t, jnp.bfloat16)
    h2 = _combine(agg1, v1, cntp, b1)

    u2 = _mm_chunked(h2, wl2_t)
    agg2 = _sc_segsum(u2, src2, dst2, z128, nch=Wl2.shape[0] // LANES)
    v2 = _mm_chunked(h2, wr2_t, jnp.bfloat16)
    h3, s, mz = _combine_att(agg2, v2, cntp, b2, Wa, bar)

    return _final(h3, s, mz, wf_t, bfr)


# speedup vs baseline: 1.0441x; 1.0001x over previous
import jax
import jax.numpy as jnp
from jax import lax
from jax.experimental import pallas as pl
from jax.experimental.pallas import tpu as pltpu
from jax.experimental.pallas import tpu_sc as plsc

F32 = jnp.float32
LANES = 128
BATCH = 64
RING = 2
BN = 400


def _sc_counts(dst2, z128, ones128, n):
    nb = dst2.shape[0] // 16
    half = (nb + 1) // 2
    n_acc = z128.shape[0] * 16
    zr = n_acc // 16
    cr = (n // 16) // 8 * 8
    tail = n - cr * 16

    mesh = plsc.VectorSubcoreMesh(core_axis_name="c", subcore_axis_name="s")

    def body(dst_ref, z_ref, ones_ref, cnt_ref, dst_v, ones_v, cacc):
        cid = lax.axis_index("c")
        sid = lax.axis_index("s")
        pltpu.sync_copy(dst_ref.at[pl.ds(sid * nb, nb)], dst_v)
        pltpu.sync_copy(ones_ref, ones_v)
        pltpu.sync_copy(z_ref, cacc.at[pl.ds(sid * zr, zr)])
        plsc.subcore_barrier()

        def cbody(b, carry):
            pltpu.sync_copy(ones_v, cacc.at[dst_v.at[b]], add=True)
            return carry
        lax.fori_loop(cid * half, half + cid * (nb - half), cbody, 0)
        plsc.subcore_barrier()
        pltpu.sync_copy(cacc.at[pl.ds(sid * cr, cr)],
                        cnt_ref.at[cid].at[pl.ds(sid * cr, cr)])
        if tail:
            @pl.when(sid == 0)
            def _():
                pltpu.sync_copy(cacc.at[pl.ds(cr * 16, tail)],
                                cnt_ref.at[cid].at[pl.ds(cr * 16, tail)])

    fn = pl.kernel(
        body,
        out_type=jax.ShapeDtypeStruct((2, n, LANES), F32),
        mesh=mesh,
        scratch_types=[
            pltpu.VMEM((nb, BATCH), jnp.int32),
            pltpu.VMEM((BATCH, LANES), F32),
            pltpu.VMEM_SHARED((n_acc, LANES), F32),
        ])
    return fn(dst2, z128, ones128)


def _sc_segsum(u, src2, dst2, z128, nch):
    n = u.shape[1]
    nb = src2.shape[0] // 16
    n_acc = z128.shape[0] * 16
    zr = n_acc // 16
    cr = (n // 16) // 8 * 8
    tail = n - cr * 16

    mesh = plsc.VectorSubcoreMesh(core_axis_name="c", subcore_axis_name="s")

    nbh = nb // 2

    def body(u_ref, src_ref, dst_ref, z128_ref, agg_ref,
             src_v, dst_v, *rest):
        bufs = rest[:RING]
        acc = rest[RING]
        sems = rest[RING + 1:]
        cid = lax.axis_index("c")
        sid = lax.axis_index("s")

        for ch in range(nch):
            @pl.when(cid == (ch % 2))
            def _():
                pltpu.sync_copy(z128_ref, acc.at[pl.ds(sid * zr, zr)])
                plsc.subcore_barrier()
                for h in range(2):
                    base = sid * nb + h * nbh
                    pltpu.sync_copy(src_ref.at[pl.ds(base, nbh)], src_v)
                    pltpu.sync_copy(dst_ref.at[pl.ds(base, nbh)], dst_v)
                    for j in range(RING):
                        pltpu.make_async_copy(u_ref.at[ch].at[src_v.at[j]],
                                              bufs[j], sems[j]).start()

                    @pl.loop(0, nbh, step=RING)
                    def _(i):
                        for j in range(RING):
                            b = i + j
                            pltpu.make_async_copy(
                                u_ref.at[ch].at[src_v.at[b]],
                                bufs[j], sems[j]).wait()
                            pltpu.sync_copy(bufs[j], acc.at[dst_v.at[b]],
                                            add=True)

                            @pl.when(b + RING < nbh)
                            def _():
                                pltpu.make_async_copy(
                                    u_ref.at[ch].at[src_v.at[b + RING]],
                                    bufs[j], sems[j]).start()
                plsc.subcore_barrier()
                pltpu.sync_copy(acc.at[pl.ds(sid * cr, cr)],
                                agg_ref.at[ch].at[pl.ds(sid * cr, cr)])
                if tail:
                    @pl.when(sid == 0)
                    def _():
                        pltpu.sync_copy(acc.at[pl.ds(cr * 16, tail)],
                                        agg_ref.at[ch].at[pl.ds(cr * 16, tail)])
                plsc.subcore_barrier()

    fn = pl.kernel(
        body,
        out_type=jax.ShapeDtypeStruct((nch, n, LANES), F32),
        mesh=mesh,
        scratch_types=(
            [pltpu.VMEM((nb // 2, BATCH), jnp.int32),
             pltpu.VMEM((nb // 2, BATCH), jnp.int32)]
            + [pltpu.VMEM((BATCH, LANES), F32)] * RING
            + [pltpu.VMEM_SHARED((n_acc, LANES), F32)]
            + [pltpu.SemaphoreType.DMA] * RING))
    return fn(u, src2, dst2, z128)


def _dot_chunks(chunks, w_ref, m):
    acc = None
    for j, a in enumerate(chunks):
        p = jnp.dot(a.astype(jnp.bfloat16),
                    w_ref[pl.ds(j * LANES, LANES), :].astype(jnp.bfloat16),
                    preferred_element_type=F32)
        acc = p if acc is None else acc + p
    return acc


def _inv_deg(cnt_ref):
    cnt = cnt_ref[0, :, 0] + cnt_ref[1, :, 0]
    return (1.0 / jnp.maximum(cnt, 1.0))[:, None]


def _mm_mean(aggc, cntp, vc, w_t, b):
    kc, n, _ = aggc.shape
    m = w_t.shape[1]
    mc = m // LANES

    def body(agg_ref, cnt_ref, v_ref, w_ref, b_ref, o_ref):
        inv = _inv_deg(cnt_ref)
        chunks = [agg_ref[j] * inv for j in range(kc)]
        acc = _dot_chunks(chunks, w_ref, m)
        for jj in range(mc):
            o_ref[jj] = jnp.maximum(
                acc[:, jj * LANES:(jj + 1) * LANES]
                + v_ref[jj].astype(F32)
                + b_ref[0][None, jj * LANES:(jj + 1) * LANES],
                0.0).astype(jnp.bfloat16)

    return pl.pallas_call(
        body,
        grid=(n // BN,),
        in_specs=[
            pl.BlockSpec((kc, BN, LANES), lambda i: (0, i, 0)),
            pl.BlockSpec((2, BN, LANES), lambda i: (0, i, 0)),
            pl.BlockSpec((mc, BN, LANES), lambda i: (0, i, 0)),
            pl.BlockSpec(w_t.shape, lambda i: (0, 0)),
            pl.BlockSpec(b.shape, lambda i: (0, 0)),
        ],
        out_specs=pl.BlockSpec((mc, BN, LANES), lambda i: (0, i, 0)),
        out_shape=jax.ShapeDtypeStruct((mc, n, LANES), jnp.bfloat16),
    )(aggc, cntp, vc, w_t, b)


def _mm_chunked(ac, w_t, out_dtype=F32):
    kc, n, _ = ac.shape
    m = w_t.shape[1]
    mc = m // LANES

    def body(a_ref, w_ref, o_ref):
        acc = _dot_chunks([a_ref[j] for j in range(kc)], w_ref, m)
        acc = acc.astype(out_dtype)
        for jj in range(mc):
            o_ref[jj] = acc[:, jj * LANES:(jj + 1) * LANES]

    return pl.pallas_call(
        body,
        grid=(n // BN,),
        in_specs=[
            pl.BlockSpec((kc, BN, LANES), lambda i: (0, i, 0)),
            pl.BlockSpec(w_t.shape, lambda i: (0, 0)),
        ],
        out_specs=pl.BlockSpec((mc, BN, LANES), lambda i: (0, i, 0)),
        out_shape=jax.ShapeDtypeStruct((mc, n, LANES), out_dtype),
    )(ac, w_t)


def _combine(aggc, vc, cntp, b):
    nch, n, _ = aggc.shape

    def body(agg_ref, v_ref, cnt_ref, b_ref, o_ref):
        inv = _inv_deg(cnt_ref)
        for j in range(nch):
            o_ref[j] = jnp.maximum(
                agg_ref[j] * inv + v_ref[j].astype(F32)
                + b_ref[0][None, j * LANES:(j + 1) * LANES],
                0.0).astype(jnp.bfloat16)

    return pl.pallas_call(
        body,
        grid=(n // BN,),
        in_specs=[
            pl.BlockSpec((nch, BN, LANES), lambda i: (0, i, 0)),
            pl.BlockSpec((nch, BN, LANES), lambda i: (0, i, 0)),
            pl.BlockSpec((2, BN, LANES), lambda i: (0, i, 0)),
            pl.BlockSpec(b.shape, lambda i: (0, 0)),
        ],
        out_specs=pl.BlockSpec((nch, BN, LANES), lambda i: (0, i, 0)),
        out_shape=jax.ShapeDtypeStruct((nch, n, LANES), jnp.bfloat16),
    )(aggc, vc, cntp, b)


def _combine_att(aggc, vc, cntp, b, wa, ba):
    nch, n, _ = aggc.shape
    nblocks = n // BN

    def body(agg_ref, v_ref, cnt_ref, b_ref, wa_ref, ba_ref,
             h_ref, s_ref, mz_ref, m_sc, z_sc):
        i = pl.program_id(0)

        @pl.when(i == 0)
        def _():
            m_sc[0] = -jnp.inf
            z_sc[0] = 0.0

        inv = _inv_deg(cnt_ref)
        sdot = None
        for j in range(nch):
            h = jnp.maximum(
                agg_ref[j] * inv + v_ref[j].astype(F32)
                + b_ref[0][None, j * LANES:(j + 1) * LANES],
                0.0)
            h_ref[j] = h
            p = jnp.sum(h * wa_ref[0][None, j * LANES:(j + 1) * LANES], axis=1)
            sdot = p if sdot is None else sdot + p
        s = jnp.tanh(sdot + ba_ref[0, 0])
        s_ref[...] = s[:, None]

        m_old = m_sc[0]
        m_new = jnp.maximum(m_old, jnp.max(s))
        z_sc[0] = z_sc[0] * jnp.exp(m_old - m_new) + jnp.sum(jnp.exp(s - m_new))
        m_sc[0] = m_new

        @pl.when(i == nblocks - 1)
        def _():
            mz_ref[0, 0] = m_sc[0]
            mz_ref[1, 0] = z_sc[0]

    return pl.pallas_call(
        body,
        grid=(nblocks,),
        in_specs=[
            pl.BlockSpec((nch, BN, LANES), lambda i: (0, i, 0)),
            pl.BlockSpec((nch, BN, LANES), lambda i: (0, i, 0)),
            pl.BlockSpec((2, BN, LANES), lambda i: (0, i, 0)),
            pl.BlockSpec(b.shape, lambda i: (0, 0)),
            pl.BlockSpec(wa.shape, lambda i: (0, 0)),
            pl.BlockSpec((1, 1), lambda i: (0, 0)),
        ],
        out_specs=[
            pl.BlockSpec((nch, BN, LANES), lambda i: (0, i, 0)),
            pl.BlockSpec((BN, 1), lambda i: (i, 0)),
            pl.BlockSpec(memory_space=pltpu.SMEM),
        ],
        out_shape=[
            jax.ShapeDtypeStruct((nch, n, LANES), F32),
            jax.ShapeDtypeStruct((n, 1), F32),
            jax.ShapeDtypeStruct((2, 1), F32),
        ],
        scratch_shapes=[pltpu.SMEM((1,), F32), pltpu.SMEM((1,), F32)],
    )(aggc, vc, cntp, b, wa, ba)


def _final(h3c, s, mz, wf_t, bf):
    nch, n, _ = h3c.shape
    m = wf_t.shape[1]

    def body(h_ref, s_ref, mz_ref, w_ref, b_ref, o_ref):
        att = jnp.exp(s_ref[...] - mz_ref[0, 0]) / mz_ref[1, 0]
        chunks = [h_ref[j] * att for j in range(nch)]
        o_ref[...] = _dot_chunks(chunks, w_ref, m) + b_ref[0][None, :]

    return pl.pallas_call(
        body,
        grid=(n // BN,),
        in_specs=[
            pl.BlockSpec((nch, BN, LANES), lambda i: (0, i, 0)),
            pl.BlockSpec((BN, 1), lambda i: (i, 0)),
            pl.BlockSpec(memory_space=pltpu.SMEM),
            pl.BlockSpec(wf_t.shape, lambda i: (0, 0)),
            pl.BlockSpec(bf.shape, lambda i: (0, 0)),
        ],
        out_specs=pl.BlockSpec((BN, m), lambda i: (i, 0)),
        out_shape=jax.ShapeDtypeStruct((n, m), F32),
    )(h3c, s, mz, wf_t, bf)


def kernel(x, edge_index, Wl0, bl0, Wr0, Wl1, bl1, Wr1, Wl2, bl2, Wr2,
           Wa, ba, Wf, bf):
    n, d_in = x.shape
    e = edge_index.shape[1]

    ept = -(-e // (16 * 8 * LANES)) * 8 * LANES
    e_pad = 16 * ept
    pad = e_pad - e
    src = edge_index[0]
    dst = edge_index[1]
    if pad:
        src = jnp.concatenate([src, jnp.zeros((pad,), jnp.int32)])
        dst = jnp.concatenate(
            [dst, n + (jnp.arange(pad, dtype=jnp.int32) % 16)])
    src2 = src.reshape(-1, BATCH)
    dst2 = dst.reshape(-1, BATCH)

    n_acc = -(-(n + 16) // 128) * 128
    z128 = jnp.zeros((n_acc // 16, LANES), F32)
    ones128 = jnp.ones((BATCH, LANES), F32)

    bf16 = jnp.bfloat16
    wl0_t = Wl0.T.astype(bf16)
    wr0_t = Wr0.T.astype(bf16)
    wl1_t = Wl1.T.astype(bf16)
    wr1_t = Wr1.T.astype(bf16)
    wl2_t = Wl2.T.astype(bf16)
    wr2_t = Wr2.T.astype(bf16)
    wf_t = Wf.T.astype(bf16)
    b0 = bl0.reshape(1, -1)
    b1 = bl1.reshape(1, -1)
    b2 = bl2.reshape(1, -1)
    bfr = bf.reshape(1, -1)
    bar = ba.reshape(1, 1)

    xc = x.reshape(n, d_in // LANES, LANES).transpose(1, 0, 2)

    cntp = _sc_counts(dst2, z128, ones128, n)
    agg0 = _sc_segsum(xc, src2, dst2, z128, nch=d_in // LANES)
    v0 = _mm_chunked(xc, wr0_t, jnp.bfloat16)
    h1 = _mm_mean(agg0, cntp, v0, wl0_t, b0)

    u1 = _mm_chunked(h1, wl1_t)
    agg1 = _sc_segsum(u1, src2, dst2, z128, nch=Wl1.shape[0] // LANES)
    v1 = _mm_chunked(h1, wr1_t, jnp.bfloat16)
    h2 = _combine(agg1, v1, cntp, b1)

    u2 = _mm_chunked(h2, wl2_t)
    agg2 = _sc_segsum(u2, src2, dst2, z128, nch=Wl2.shape[0] // LANES)
    v2 = _mm_chunked(h2, wr2_t, jnp.bfloat16)
    h3, s, mz = _combine_att(agg2, v2, cntp, b2, Wa, bar)

    return _final(h3, s, mz, wf_t, bfr)
